# baseline (device time: 52365 ns/iter reference)
import jax
import jax.numpy as jnp
from jax import lax
from jax.experimental import pallas as pl
from jax.experimental.pallas import tpu as pltpu

B, H, D = 16, 16, 64
HD = H * D
HB = 2
HBD = HB * D
NH = H // HB
SCALE = D ** -0.5


def kernel(Q, K, V):
    b, kseq, h, d = K.shape

    def body(q_ref, k_ref, v_ref, o_ref,
             m_ref, l_ref, acc_ref, qs_ref, comm_ref, stats_ref,
             send_sems, recv_sems):
        hi = pl.program_id(0)
        my_x = lax.axis_index("x")
        my_y = lax.axis_index("y")
        my_z = lax.axis_index("z")
        partner = (1 - my_x, my_y, my_z)

        rowi = jax.lax.broadcasted_iota(jnp.int32, (HB, HBD), 0)
        coli = jax.lax.broadcasted_iota(jnp.int32, (HB, HBD), 1) // D
        E2 = (rowi == coli).astype(jnp.float32)

        @pl.when(hi == 0)
        def _first_step():
            qs_ref[...] = q_ref[:, 0, :, :][..., None] * SCALE
            barrier = pltpu.get_barrier_semaphore()
            pl.semaphore_signal(
                barrier, inc=1,
                device_id=partner, device_id_type=pl.DeviceIdType.MESH,
            )
            pl.semaphore_wait(barrier, 1)

        qs = qs_ref[:, pl.ds(hi * HB, HB), :, :]
        s = jnp.sum(k_ref[...] * qs, axis=2)
        m = jnp.max(s, axis=-1)
        p = jnp.exp(s - m[..., None])
        l = jnp.sum(p, axis=-1)
        vt = v_ref[...].reshape(B, HBD, kseq)
        o_full = lax.dot_general(
            p.astype(jnp.bfloat16), vt.astype(jnp.bfloat16),
            dimension_numbers=(((2,), (2,)), ((0,), (0,))),
            preferred_element_type=jnp.float32,
        )
        pv = jnp.sum(o_full * E2[None, :, :], axis=1)
        acc_ref[:, pl.ds(hi * HBD, HBD)] = pv
        m_ref[hi] = m
        l_ref[hi] = l

        @pl.when(hi == NH - 1)
        def _exchange_and_finish():
            comm_ref[0] = acc_ref[...]
            stats_ref[0, 0] = m_ref[...]
            stats_ref[0, 1] = l_ref[...]
            rdma_acc = pltpu.make_async_remote_copy(
                src_ref=comm_ref.at[0],
                dst_ref=comm_ref.at[1],
                send_sem=send_sems.at[0],
                recv_sem=recv_sems.at[0],
                device_id=partner,
                device_id_type=pl.DeviceIdType.MESH,
            )
            rdma_stats = pltpu.make_async_remote_copy(
                src_ref=stats_ref.at[0],
                dst_ref=stats_ref.at[1],
                send_sem=send_sems.at[1],
                recv_sem=recv_sems.at[1],
                device_id=partner,
                device_id_type=pl.DeviceIdType.MESH,
            )
            rdma_acc.start()
            rdma_stats.start()
            rdma_acc.wait()
            rdma_stats.wait()

            for hb in range(NH):
                cols = pl.ds(hb * HBD, HBD)
                m_mine = m_ref[hb]
                l_mine = l_ref[hb]
                m_p = stats_ref[1, 0, hb]
                l_p = stats_ref[1, 1, hb]
                m_tot = jnp.maximum(m_mine, m_p)
                a_mine = jnp.exp(m_mine - m_tot)
                a_p = jnp.exp(m_p - m_tot)
                den = l_mine * a_mine + l_p * a_p

                def expand(x):
                    return lax.dot_general(
                        x, E2,
                        dimension_numbers=(((1,), (0,)), ((), ())),
                        preferred_element_type=jnp.float32,
                    )

                num = (acc_ref[:, cols] * expand(a_mine)
                       + comm_ref[1, :, cols] * expand(a_p))
                o_ref[:, cols] = num / expand(den)

    kt = jnp.transpose(K, (0, 2, 3, 1))
    vt = jnp.transpose(V, (0, 2, 3, 1))

    out = pl.pallas_call(
        body,
        grid=(NH,),
        out_shape=jax.ShapeDtypeStruct((B, HD), jnp.float32),
        in_specs=[
            pl.BlockSpec((B, 1, H, D), lambda hi: (0, 0, 0, 0)),
            pl.BlockSpec((B, HB, D, kseq), lambda hi: (0, hi, 0, 0)),
            pl.BlockSpec((B, HB, D, kseq), lambda hi: (0, hi, 0, 0)),
        ],
        out_specs=pl.BlockSpec((B, HD), lambda hi: (0, 0)),
        scratch_shapes=[
            pltpu.VMEM((NH, B, HB), jnp.float32),
            pltpu.VMEM((NH, B, HB), jnp.float32),
            pltpu.VMEM((B, HD), jnp.float32),
            pltpu.VMEM((B, H, D, 1), jnp.float32),
            pltpu.VMEM((2, B, HD), jnp.float32),
            pltpu.VMEM((2, 2, NH, B, HB), jnp.float32),
            pltpu.SemaphoreType.DMA((2,)),
            pltpu.SemaphoreType.DMA((2,)),
        ],
        compiler_params=pltpu.CompilerParams(
            collective_id=0, vmem_limit_bytes=48 * 1024 * 1024
        ),
    )(Q, kt, vt)
    return out.reshape(B, 1, H, D)
